# Initial kernel scaffold; baseline (speedup 1.0000x reference)
import jax
import jax.numpy as jnp
from jax.experimental import pallas as pl


def kernel(patches, pos_table):
    raise NotImplementedError("write your pallas kernel here")


# speedup vs baseline: 1.8019x; 1.8019x over previous
import jax
import jax.numpy as jnp
from jax.experimental import pallas as pl

B = 4
N_P = 8192
D = 768
PB = 512


def _add_kernel(patches_ref, pos_ref, out_ref):
    out_ref[...] = patches_ref[...] + pos_ref[...][None, :, :]


def kernel(patches, pos_table):
    grid = (N_P // PB,)
    return pl.pallas_call(
        _add_kernel,
        grid=grid,
        in_specs=[
            pl.BlockSpec((B, PB, D), lambda i: (0, i, 0)),
            pl.BlockSpec((PB, D), lambda i: (i, 0)),
        ],
        out_specs=pl.BlockSpec((B, PB, D), lambda i: (0, i, 0)),
        out_shape=jax.ShapeDtypeStruct((B, N_P, D), jnp.float32),
    )(patches, pos_table)


# speedup vs baseline: 1.8084x; 1.0036x over previous
import jax
import jax.numpy as jnp
from jax.experimental import pallas as pl

B = 4
N_P = 8192
D = 768
PB = 1024


def _add_kernel(patches_ref, pos_ref, out_ref):
    out_ref[...] = patches_ref[...] + pos_ref[...][None, :, :]


def kernel(patches, pos_table):
    grid = (N_P // PB,)
    return pl.pallas_call(
        _add_kernel,
        grid=grid,
        in_specs=[
            pl.BlockSpec((B, PB, D), lambda i: (0, i, 0)),
            pl.BlockSpec((PB, D), lambda i: (i, 0)),
        ],
        out_specs=pl.BlockSpec((B, PB, D), lambda i: (0, i, 0)),
        out_shape=jax.ShapeDtypeStruct((B, N_P, D), jnp.float32),
    )(patches, pos_table)
